# baseline (device time: 27236 ns/iter reference)
import jax
import jax.numpy as jnp
from jax import lax
from jax.experimental import pallas as pl
from jax.experimental.pallas import tpu as pltpu

N_DEV = 4
CHUNK = 2


def kernel(x, w_mat):
    m, _ = x.shape
    _, n = w_mat.shape
    f32 = jnp.float32
    bf16 = jnp.bfloat16
    ncol = n // CHUNK

    def body(x_ref, w_ref, out_ref,
             s1, r1, s2, r2, g2s, g2r, send_sems, recv_sems):
        my = lax.axis_index("i")
        pA = my + 1 - 2 * (my % 2)
        pB = 3 - my

        barrier_sem = pltpu.get_barrier_semaphore()
        for nbr in (pA, pB):
            pl.semaphore_signal(
                barrier_sem, inc=1,
                device_id=(nbr,), device_id_type=pl.DeviceIdType.MESH,
            )

        kA = jnp.where((my == 1) | (my == 2), 1, 0)
        kB = my // 2
        qA = my // 2
        qB = my % 2
        baseA = kA * 256
        baseB = 512 + kB * 256

        dsets = (
            ((pA, pB, pB, pA), baseA, qA, (1 - kA) * 256),
            ((pB, pA, pA, pB), baseB, qB, 512 + (1 - kB) * 256),
        )

        def rdma(p, d, c, src, dst):
            op = pltpu.make_async_remote_copy(
                src_ref=src, dst_ref=dst,
                send_sem=send_sems.at[p, d, c],
                recv_sem=recv_sems.at[p, d, c],
                device_id=(dsets[d][0][p],),
                device_id_type=pl.DeviceIdType.MESH,
            )
            op.start()
            return op

        pending = {}
        for c in range(CHUNK):
            cs = slice(c * ncol, (c + 1) * ncol)
            for d, (_, base, q, sent0) in enumerate(dsets):
                s1[d, :, cs] = jnp.dot(
                    x_ref[pl.ds(sent0, 256), :], w_ref[:, cs],
                    preferred_element_type=f32).astype(bf16)
            if c == 0:
                pl.semaphore_wait(barrier_sem, 2)
            ops = [rdma(0, d, c, s1.at[d, :, cs], r1.at[d, :, cs])
                   for d in range(2)]
            for d, (_, base, q, sent0) in enumerate(dsets):
                out_ref[pl.ds(base, 256), cs] = jnp.dot(
                    x_ref[pl.ds(base, 256), :], w_ref[:, cs],
                    preferred_element_type=f32)
            pending[c] = ops

        for c in range(CHUNK):
            cs = slice(c * ncol, (c + 1) * ncol)
            for op in pending[c]:
                op.wait()
            ops = []
            for d, (_, base, q, sent0) in enumerate(dsets):
                fq = (1 - q) * 128
                s2[d, :, cs] = (
                    out_ref[pl.ds(base + fq, 128), cs]
                    + r1[d, pl.ds(fq, 128), cs].astype(f32)).astype(bf16)
                ops.append(rdma(1, d, c, s2.at[d, :, cs], r2.at[d, :, cs]))
                own = base + q * 128
                out_ref[pl.ds(own, 128), cs] = (
                    out_ref[pl.ds(own, 128), cs]
                    + r1[d, pl.ds(q * 128, 128), cs].astype(f32))
            pending[c] = ops

        for c in range(CHUNK):
            cs = slice(c * ncol, (c + 1) * ncol)
            for op in pending[c]:
                op.wait()
            ops = []
            for d, (_, base, q, sent0) in enumerate(dsets):
                own = base + q * 128
                y = (out_ref[pl.ds(own, 128), cs]
                     + r2[d, :, cs].astype(f32))
                y = y * (1.0 / (1.0 + jnp.exp(-y)))
                out_ref[pl.ds(own, 128), cs] = y
                qs = pl.ds(q * 128, 128)
                g2s[d, qs, cs] = y.astype(bf16)
                ops.append(rdma(2, d, c, g2s.at[d, qs, cs],
                                g2s.at[d, qs, cs]))
            pending[c] = ops

        for c in range(CHUNK):
            cs = slice(c * ncol, (c + 1) * ncol)
            for op in pending[c]:
                op.wait()
            ops = []
            for d in range(2):
                ops.append(rdma(3, d, c, g2s.at[d, :, cs],
                                g2r.at[d, :, cs]))
            for d, (_, base, q, sent0) in enumerate(dsets):
                rq = pl.ds((1 - q) * 128, 128)
                out_ref[pl.ds(base + (1 - q) * 128, 128), cs] = (
                    g2s[d, rq, cs].astype(f32))
            pending[c] = ops

        for c in range(CHUNK):
            cs = slice(c * ncol, (c + 1) * ncol)
            for op in pending[c]:
                op.wait()
            out_ref[pl.ds((1 - kA) * 256, 256), cs] = (
                g2r[0, :, cs].astype(f32))
            out_ref[pl.ds(512 + (1 - kB) * 256, 256), cs] = (
                g2r[1, :, cs].astype(f32))

    return pl.pallas_call(
        body,
        out_shape=jax.ShapeDtypeStruct((m, n), f32),
        in_specs=[
            pl.BlockSpec(memory_space=pltpu.VMEM),
            pl.BlockSpec(memory_space=pltpu.VMEM),
        ],
        out_specs=pl.BlockSpec(memory_space=pltpu.VMEM),
        scratch_shapes=[
            pltpu.VMEM((2, 256, n), bf16),
            pltpu.VMEM((2, 256, n), bf16),
            pltpu.VMEM((2, 128, n), bf16),
            pltpu.VMEM((2, 128, n), bf16),
            pltpu.VMEM((2, 256, n), bf16),
            pltpu.VMEM((2, 256, n), bf16),
            pltpu.SemaphoreType.DMA((4, 2, CHUNK)),
            pltpu.SemaphoreType.DMA((4, 2, CHUNK)),
        ],
        compiler_params=pltpu.CompilerParams(collective_id=0),
    )(x, w_mat)


# device time: 25962 ns/iter; 1.0491x vs baseline; 1.0491x over previous
import jax
import jax.numpy as jnp
from jax import lax
from jax.experimental import pallas as pl
from jax.experimental.pallas import tpu as pltpu

N_DEV = 4
CHUNK = 4


def kernel(x, w_mat):
    m, _ = x.shape
    _, n = w_mat.shape
    f32 = jnp.float32
    bf16 = jnp.bfloat16
    ncol = n // CHUNK

    def body(x_ref, w_ref, out_ref,
             s1, r1, s2, r2, g2s, g2r, send_sems, recv_sems):
        my = lax.axis_index("i")
        pA = my + 1 - 2 * (my % 2)
        pB = 3 - my

        barrier_sem = pltpu.get_barrier_semaphore()
        for nbr in (pA, pB):
            pl.semaphore_signal(
                barrier_sem, inc=1,
                device_id=(nbr,), device_id_type=pl.DeviceIdType.MESH,
            )

        kA = jnp.where((my == 1) | (my == 2), 1, 0)
        kB = my // 2
        qA = my // 2
        qB = my % 2
        baseA = kA * 256
        baseB = 512 + kB * 256

        dsets = (
            ((pA, pB, pB, pA), baseA, qA, (1 - kA) * 256),
            ((pB, pA, pA, pB), baseB, qB, 512 + (1 - kB) * 256),
        )

        def rdma(p, d, c, src, dst):
            op = pltpu.make_async_remote_copy(
                src_ref=src, dst_ref=dst,
                send_sem=send_sems.at[p, d, c],
                recv_sem=recv_sems.at[p, d, c],
                device_id=(dsets[d][0][p],),
                device_id_type=pl.DeviceIdType.MESH,
            )
            op.start()
            return op

        pending = {}
        for c in range(CHUNK):
            cs = slice(c * ncol, (c + 1) * ncol)
            for d, (_, base, q, sent0) in enumerate(dsets):
                s1[d, :, cs] = jnp.dot(
                    x_ref[pl.ds(sent0, 256), :], w_ref[:, cs],
                    preferred_element_type=f32).astype(bf16)
            if c == 0:
                pl.semaphore_wait(barrier_sem, 2)
            ops = [rdma(0, d, c, s1.at[d, :, cs], r1.at[d, :, cs])
                   for d in range(2)]
            for d, (_, base, q, sent0) in enumerate(dsets):
                out_ref[pl.ds(base, 256), cs] = jnp.dot(
                    x_ref[pl.ds(base, 256), :], w_ref[:, cs],
                    preferred_element_type=f32)
            pending[c] = ops

        for c in range(CHUNK):
            cs = slice(c * ncol, (c + 1) * ncol)
            ops = []
            for d, (_, base, q, sent0) in enumerate(dsets):
                pending[c][d].wait()
                fq = (1 - q) * 128
                s2[d, :, cs] = (
                    out_ref[pl.ds(base + fq, 128), cs]
                    + r1[d, pl.ds(fq, 128), cs].astype(f32)).astype(bf16)
                ops.append(rdma(1, d, c, s2.at[d, :, cs], r2.at[d, :, cs]))
                own = base + q * 128
                out_ref[pl.ds(own, 128), cs] = (
                    out_ref[pl.ds(own, 128), cs]
                    + r1[d, pl.ds(q * 128, 128), cs].astype(f32))
            pending[c] = ops

        for c in range(CHUNK):
            cs = slice(c * ncol, (c + 1) * ncol)
            ops = []
            for d, (_, base, q, sent0) in enumerate(dsets):
                pending[c][d].wait()
                own = base + q * 128
                y = (out_ref[pl.ds(own, 128), cs]
                     + r2[d, :, cs].astype(f32))
                y = y * (1.0 / (1.0 + jnp.exp(-y)))
                out_ref[pl.ds(own, 128), cs] = y
                qs = pl.ds(q * 128, 128)
                g2s[d, qs, cs] = y.astype(bf16)
                ops.append(rdma(2, d, c, g2s.at[d, qs, cs],
                                g2s.at[d, qs, cs]))
            pending[c] = ops

        for c in range(CHUNK):
            cs = slice(c * ncol, (c + 1) * ncol)
            ops = []
            for d, (_, base, q, sent0) in enumerate(dsets):
                pending[c][d].wait()
                ops.append(rdma(3, d, c, g2s.at[d, :, cs],
                                g2r.at[d, :, cs]))
                rq = pl.ds((1 - q) * 128, 128)
                out_ref[pl.ds(base + (1 - q) * 128, 128), cs] = (
                    g2s[d, rq, cs].astype(f32))
            pending[c] = ops

        for c in range(CHUNK):
            cs = slice(c * ncol, (c + 1) * ncol)
            for d, off in ((0, (1 - kA) * 256), (1, 512 + (1 - kB) * 256)):
                pending[c][d].wait()
                out_ref[pl.ds(off, 256), cs] = g2r[d, :, cs].astype(f32)

    return pl.pallas_call(
        body,
        out_shape=jax.ShapeDtypeStruct((m, n), f32),
        in_specs=[
            pl.BlockSpec(memory_space=pltpu.VMEM),
            pl.BlockSpec(memory_space=pltpu.VMEM),
        ],
        out_specs=pl.BlockSpec(memory_space=pltpu.VMEM),
        scratch_shapes=[
            pltpu.VMEM((2, 256, n), bf16),
            pltpu.VMEM((2, 256, n), bf16),
            pltpu.VMEM((2, 128, n), bf16),
            pltpu.VMEM((2, 128, n), bf16),
            pltpu.VMEM((2, 256, n), bf16),
            pltpu.VMEM((2, 256, n), bf16),
            pltpu.SemaphoreType.DMA((4, 2, CHUNK)),
            pltpu.SemaphoreType.DMA((4, 2, CHUNK)),
        ],
        compiler_params=pltpu.CompilerParams(collective_id=0),
    )(x, w_mat)
